# baseline (speedup 1.0000x reference)
import functools

import jax
import jax.numpy as jnp
from jax import lax
from jax.experimental import pallas as pl
from jax.experimental.pallas import tpu as pltpu
from jax.experimental.pallas import tpu_sc as plsc

_NUM_NEG = 16
_NC = 2
_NS = 16
_NW = _NC * _NS
_LANE = 128


def _sc_gather_build(n_tot, chunks_per_w):
    mesh = plsc.VectorSubcoreMesh(core_axis_name="c", subcore_axis_name="s")

    @functools.partial(
        pl.kernel,
        out_type=jax.ShapeDtypeStruct((_NW, chunks_per_w, _LANE), jnp.float32),
        mesh=mesh,
        scratch_types=[
            pltpu.VMEM((chunks_per_w, _LANE), jnp.int32),
            pltpu.VMEM((chunks_per_w, _LANE), jnp.float32),
            pltpu.SemaphoreType.DMA,
        ],
    )
    def sc_gather(yhat_hbm, idx_hbm, out_hbm, idx_v, vals_v, sem):
        wid = lax.axis_index("s") * _NC + lax.axis_index("c")
        pltpu.sync_copy(idx_hbm.at[wid], idx_v)

        def fire(j, carry):
            pltpu.async_copy(yhat_hbm.at[idx_v.at[j]], vals_v.at[j], sem).wait()
            return carry

        lax.fori_loop(0, chunks_per_w, fire, 0)
        pltpu.sync_copy(vals_v, out_hbm.at[wid])

    return sc_gather


def _tc_loss_body(vals_ref, w_ref, c_ref, out_ref):
    i = pl.program_id(0)
    x = vals_ref[...]
    w = w_ref[...]
    c = c_ref[...]
    sp = jnp.maximum(x, 0.0) + jnp.log1p(jnp.exp(-jnp.abs(x)))
    part = jnp.sum(sp * w - x * c)

    @pl.when(i == 0)
    def _():
        out_ref[0, 0] = 0.0

    out_ref[0, 0] += part


def kernel(y_hat, y_pos):
    B, I = y_hat.shape
    T1 = y_pos.shape[1]
    n_idx = T1 * (1 + _NUM_NEG)
    per_row = -(-n_idx // _LANE) * _LANE
    n_tot = B * per_row
    per_w = n_tot // _NW
    chunks_per_w = per_w // _LANE

    neg_key = jax.random.key(42)
    y_neg = jax.random.randint(neg_key, (B, T1, _NUM_NEG), 0, I, dtype=jnp.int64)

    idx = jnp.concatenate(
        [y_pos.astype(jnp.int32), y_neg.reshape(B, -1).astype(jnp.int32)], axis=1
    )
    idx = jnp.pad(idx, ((0, 0), (0, per_row - n_idx)))
    flat_idx = (idx + jnp.arange(B, dtype=jnp.int32)[:, None] * I).reshape(
        _NW, chunks_per_w, _LANE
    )

    vals = _sc_gather_build(n_tot, chunks_per_w)(y_hat.reshape(-1), flat_idx)
    vals2d = vals.reshape(B, per_row)

    wp = 1.0 / (B * T1)
    wn = 1.0 / (B * T1 * _NUM_NEG)
    w_vec = jnp.concatenate(
        [
            jnp.full((T1,), wp, jnp.float32),
            jnp.full((n_idx - T1,), wn, jnp.float32),
            jnp.zeros((per_row - n_idx,), jnp.float32),
        ]
    ).reshape(1, per_row)
    c_vec = jnp.concatenate(
        [
            jnp.full((T1,), wp, jnp.float32),
            jnp.zeros((per_row - T1,), jnp.float32),
        ]
    ).reshape(1, per_row)

    bm = 256
    loss = pl.pallas_call(
        _tc_loss_body,
        grid=(B // bm,),
        in_specs=[
            pl.BlockSpec((bm, per_row), lambda i: (i, 0)),
            pl.BlockSpec((1, per_row), lambda i: (0, 0)),
            pl.BlockSpec((1, per_row), lambda i: (0, 0)),
        ],
        out_specs=pl.BlockSpec(memory_space=pltpu.SMEM),
        out_shape=jax.ShapeDtypeStruct((1, 1), jnp.float32),
    )(vals2d, w_vec, c_vec)
    return loss.reshape(())


# speedup vs baseline: 1.8829x; 1.8829x over previous
import functools

import jax
import jax.numpy as jnp
import numpy as np
from jax import lax
from jax.experimental import pallas as pl
from jax.experimental.pallas import tpu as pltpu
from jax.experimental.pallas import tpu_sc as plsc

_NUM_NEG = 16
_B, _I, _T1 = 1024, 100000, 20
_NC, _NS = 2, 16
_NW = _NC * _NS
_LANE = 128
_ROWS_PER_W = _B // _NW
_NPOS_W = _ROWS_PER_W * _T1
_POS_ROUNDS = _NPOS_W // 16


def _rotl(x, r):
    return ((x << np.uint32(r)) | (x >> np.uint32(32 - r))).astype(np.uint32)


def _tf_hash(k1, k2, x0, x1):
    with np.errstate(over="ignore"):
        ks0, ks1 = np.uint32(k1), np.uint32(k2)
        ks2 = np.uint32(ks0 ^ ks1 ^ np.uint32(0x1BD11BDA))
        x0 = (x0 + ks0).astype(np.uint32)
        x1 = (x1 + ks1).astype(np.uint32)
        rot0, rot1 = (13, 15, 26, 6), (17, 29, 16, 24)

        def rounds(x0, x1, rots):
            for r in rots:
                x0 = (x0 + x1).astype(np.uint32)
                x1 = _rotl(x1, r)
                x1 = (x1 ^ x0).astype(np.uint32)
            return x0, x1

        for i, (rots, ka, kb) in enumerate(
            [(rot0, ks1, ks2), (rot1, ks2, ks0), (rot0, ks0, ks1),
             (rot1, ks1, ks2), (rot0, ks2, ks0)]
        ):
            x0, x1 = rounds(x0, x1, rots)
            x0 = (x0 + ka).astype(np.uint32)
            x1 = (x1 + kb + np.uint32(i + 1)).astype(np.uint32)
    return x0, x1


def _sample_neg_host():
    b1, b2 = _tf_hash(
        np.uint32(0), np.uint32(42),
        np.array([0, 0], np.uint32), np.array([0, 1], np.uint32),
    )
    n = _B * _T1 * _NUM_NEG
    hb1, hb2 = _tf_hash(
        b1[1], b2[1], np.zeros(n, np.uint32), np.arange(n, dtype=np.uint32)
    )
    bits = (hb1 ^ hb2).astype(np.uint32)
    return (bits % np.uint32(_I)).astype(np.int64).reshape(_B, _T1, _NUM_NEG)


def _build_neg_routing():
    yneg = _sample_neg_host()
    cols = yneg.reshape(-1)
    rows = np.repeat(np.arange(_B), _T1 * _NUM_NEG)
    order = np.argsort(cols // _LANE, kind="stable")
    rows, cols = rows[order], cols[order]
    wins = cols // _LANE
    lanes = cols % _LANE

    chunk_rows, chunk_lanes, chunk_wins, chunk_valid = [], [], [], []
    start, n = 0, len(cols)
    while start < n:
        j = wins[start]
        end = start
        while end < n and wins[end] == j and end - start < _LANE:
            end += 1
        cnt = end - start
        pad = _LANE - cnt
        chunk_rows.append(
            np.concatenate([rows[start:end], (np.arange(pad) * 37) % _B])
        )
        chunk_lanes.append(
            np.concatenate([lanes[start:end], np.zeros(pad, np.int64)])
        )
        chunk_wins.append(j * _LANE)
        chunk_valid.append(np.concatenate([np.ones(cnt), np.zeros(pad)]))
        start = end

    cpw = -(-len(chunk_wins) // _NW)
    if cpw % 2:
        cpw += 1
    while len(chunk_wins) < cpw * _NW:
        chunk_rows.append((np.arange(_LANE) * 53) % _B)
        chunk_lanes.append(np.zeros(_LANE, np.int64))
        chunk_wins.append(0)
        chunk_valid.append(np.zeros(_LANE))

    rows_tab = np.stack(chunk_rows).reshape(_NW, cpw * _LANE).astype(np.int32)
    lanes_tab = np.stack(chunk_lanes).reshape(_NW, cpw * _LANE).astype(np.int32)
    wins_tab = np.asarray(chunk_wins, np.int64).reshape(_NW, cpw)
    cpw16 = -(-cpw // 16) * 16
    wins16 = np.zeros((_NW, cpw16), np.int32)
    wins16[:, :cpw] = wins_tab
    valid_tab = np.stack(chunk_valid).reshape(_NW, cpw * _LANE)
    return rows_tab, lanes_tab, wins16, valid_tab, cpw, cpw16


(_NEG_ROWS, _NEG_LANES, _NEG_WINS, _NEG_VALID, _CPW, _CPW16) = (
    _build_neg_routing()
)
_NSLOT_NEG = _CPW * _LANE
_SLOTS = _NSLOT_NEG + _NPOS_W
assert (_NW * _SLOTS) % 512 == 0

_WP = 1.0 / (_B * _T1)
_WN = 1.0 / (_B * _T1 * _NUM_NEG)
_W_TAB = np.concatenate(
    [_NEG_VALID * _WN, np.full((_NW, _NPOS_W), _WP)], axis=1
).astype(np.float32)
_C_TAB = np.concatenate(
    [np.zeros((_NW, _NSLOT_NEG)), np.full((_NW, _NPOS_W), _WP)], axis=1
).astype(np.float32)


def _iota16():
    return lax.iota(jnp.int32, 16)


def _vec_at(vec_ref, t):
    base = pl.multiple_of((t // 16) * 16, 16)
    v16 = vec_ref[pl.ds(base, 16)]
    return jnp.max(jnp.where(_iota16() == (t % 16), v16, 0))


def _sc_gather_build():
    mesh = plsc.VectorSubcoreMesh(core_axis_name="c", subcore_axis_name="s")

    @functools.partial(
        pl.kernel,
        out_type=jax.ShapeDtypeStruct((_NW, _SLOTS), jnp.float32),
        mesh=mesh,
        compiler_params=pltpu.CompilerParams(needs_layout_passes=False),
        scratch_types=[
            pltpu.VMEM((_NSLOT_NEG,), jnp.int32),
            pltpu.VMEM((_NSLOT_NEG,), jnp.int32),
            pltpu.VMEM((_CPW16,), jnp.int32),
            pltpu.VMEM((_NPOS_W,), jnp.int32),
            pltpu.VMEM((_NPOS_W,), jnp.int32),
            pltpu.VMEM((_SLOTS,), jnp.float32),
            pltpu.VMEM((2 * _LANE, _LANE), jnp.float32),
            pltpu.SemaphoreType.DMA,
            pltpu.SemaphoreType.DMA,
        ],
    )
    def sc_gather(
        yhat,
        nrows_h,
        nlanes_h,
        nwins_h,
        pwins_h,
        planes_h,
        out_h,
        nrows_v,
        nlanes_v,
        nwins_v,
        pwins_v,
        planes_v,
        vals_v,
        ring_v,
        sem0,
        sem1,
    ):
        wid = lax.axis_index("s") * _NC + lax.axis_index("c")
        sems = [sem0, sem1]
        pltpu.sync_copy(nrows_h.at[wid], nrows_v)
        pltpu.sync_copy(nlanes_h.at[wid], nlanes_v)
        pltpu.sync_copy(nwins_h.at[wid], nwins_v)
        pltpu.sync_copy(pwins_h.at[wid], pwins_v)
        pltpu.sync_copy(planes_h.at[wid], planes_v)

        def _fire_neg(k, par):
            win = pl.multiple_of(_vec_at(nwins_v, k), _LANE)
            src = yhat.at[
                nrows_v.at[pl.ds(k * _LANE, _LANE)], pl.ds(win, _LANE)
            ]
            pltpu.async_copy(
                src, ring_v.at[pl.ds(par * _LANE, _LANE)], sems[par]
            )

        def _wait_ring(par):
            pltpu.make_async_copy(
                yhat.at[pl.ds(0, _LANE), pl.ds(0, _LANE)],
                ring_v.at[pl.ds(par * _LANE, _LANE)],
                sems[par],
            ).wait()

        def _extract(par, l16s, out_base):
            for u in range(8):
                t16 = _iota16() + (par * _LANE + u * 16)
                vals_v[pl.ds(out_base + u * 16, 16)] = plsc.load_gather(
                    ring_v, [t16, l16s[u]]
                )

        def _neg_l16s(k):
            return [
                nlanes_v[pl.ds(k * _LANE + u * 16, 16)] for u in range(8)
            ]

        _fire_neg(0, 0)
        _fire_neg(1, 1)

        def neg_body(kk, carry):
            k0 = 2 * kk
            _wait_ring(0)
            _extract(0, _neg_l16s(k0), k0 * _LANE)

            @pl.when(k0 + 2 < _CPW)
            def _():
                _fire_neg(k0 + 2, 0)

            _wait_ring(1)
            _extract(1, _neg_l16s(k0 + 1), (k0 + 1) * _LANE)

            @pl.when(k0 + 3 < _CPW)
            def _():
                _fire_neg(k0 + 3, 1)

            return carry

        lax.fori_loop(0, _CPW // 2, neg_body, 0)

        def _fire_pos(r, par):
            def fire_one(i, carry):
                t = r * 16 + i
                win = pl.multiple_of(_vec_at(pwins_v, t), _LANE)
                rowb = pl.multiple_of(
                    wid * _ROWS_PER_W + (t // (8 * _T1)) * 8, 8
                )
                src = yhat.at[pl.ds(rowb, 8), pl.ds(win, _LANE)]
                dst = ring_v.at[
                    pl.ds(pl.multiple_of(par * _LANE + 8 * i, 8), 8)
                ]
                pltpu.async_copy(src, dst, sems[par])
                return carry

            lax.fori_loop(0, 16, fire_one, 0)

        def _extract_pos(r, par):
            t16 = (par * _LANE) + 8 * _iota16() + lax.rem(
                lax.div(r * 16 + _iota16(), _T1), 8
            )
            l16 = planes_v[pl.ds(r * 16, 16)]
            vals_v[pl.ds(_NSLOT_NEG + r * 16, 16)] = plsc.load_gather(
                ring_v, [t16, l16]
            )

        _fire_pos(0, 0)
        _fire_pos(1, 1)

        def pos_body(rr, carry):
            r0 = 2 * rr
            _wait_ring(0)
            _extract_pos(r0, 0)

            @pl.when(r0 + 2 < _POS_ROUNDS)
            def _():
                _fire_pos(r0 + 2, 0)

            _wait_ring(1)
            _extract_pos(r0 + 1, 1)

            @pl.when(r0 + 3 < _POS_ROUNDS)
            def _():
                _fire_pos(r0 + 3, 1)

            return carry

        lax.fori_loop(0, _POS_ROUNDS // 2, pos_body, 0)

        pltpu.sync_copy(vals_v, out_h.at[wid])

    return sc_gather


def _tc_loss_body(vals_ref, w_ref, c_ref, out_ref):
    x = vals_ref[...]
    w = w_ref[...]
    c = c_ref[...]
    sp = jnp.maximum(x, 0.0) + jnp.log1p(jnp.exp(-jnp.abs(x)))
    out_ref[0, 0] = jnp.sum(sp * w - x * c)


def kernel(y_hat, y_pos):
    pos_cols = y_pos.astype(jnp.int32).reshape(_NW, _NPOS_W)
    pos_wins = (pos_cols // _LANE) * _LANE
    pos_lanes = pos_cols % _LANE

    vals = _sc_gather_build()(
        y_hat,
        jnp.asarray(_NEG_ROWS),
        jnp.asarray(_NEG_LANES),
        jnp.asarray(_NEG_WINS),
        pos_wins,
        pos_lanes,
    )

    tot = _NW * _SLOTS
    vals2d = vals.reshape(tot // 512, 512)
    w2d = jnp.asarray(_W_TAB).reshape(tot // 512, 512)
    c2d = jnp.asarray(_C_TAB).reshape(tot // 512, 512)

    loss = pl.pallas_call(
        _tc_loss_body,
        out_specs=pl.BlockSpec(memory_space=pltpu.SMEM),
        out_shape=jax.ShapeDtypeStruct((1, 1), jnp.float32),
    )(vals2d, w2d, c2d)
    return loss.reshape(())
